# baseline (speedup 1.0000x reference)
import functools

import jax
import jax.numpy as jnp
from jax.experimental import pallas as pl
from jax.experimental.pallas import tpu as pltpu


def _leaky(x):
    return jnp.where(x >= 0, x, 0.2 * x)


def _proj_body(nf_ref, w_ref, out_ref):
    out_ref[:] = jnp.dot(nf_ref[:], w_ref[:], preferred_element_type=jnp.float32)


def _node_proj(nf, wcat):
    n, k = nf.shape
    m = wcat.shape[1]
    bn = 2000
    return pl.pallas_call(
        _proj_body,
        grid=(n // bn,),
        in_specs=[
            pl.BlockSpec((bn, k), lambda i: (i, 0)),
            pl.BlockSpec((k, m), lambda i: (0, 0)),
        ],
        out_specs=pl.BlockSpec((bn, m), lambda i: (i, 0)),
        out_shape=jax.ShapeDtypeStruct((n, m), jnp.float32),
    )(nf, wcat)


def _bro_body(g_ref, ef_ref, w1e_ref, b1_ref, w2_ref, b2_ref, w3_ref, b3_ref,
              w4_ref, b4_ref, w5_ref, b5_ref, out_ref):
    h = g_ref[:] + jnp.dot(ef_ref[:], w1e_ref[:], preferred_element_type=jnp.float32) + b1_ref[:]
    h = _leaky(h)
    h = _leaky(jnp.dot(h, w2_ref[:], preferred_element_type=jnp.float32) + b2_ref[:])
    h = _leaky(jnp.dot(h, w3_ref[:], preferred_element_type=jnp.float32) + b3_ref[:])
    h = _leaky(jnp.dot(h, w4_ref[:], preferred_element_type=jnp.float32) + b4_ref[:])
    out_ref[:] = jnp.dot(h, w5_ref[:], preferred_element_type=jnp.float32) + b5_ref[:]


def _bro_edge_mlp(g, ef, w1e, b1, w2, b2, w3, b3, w4, b4, w5, b5):
    e = g.shape[0]
    be = 3200
    full = lambda a: pl.BlockSpec(a.shape, lambda i: tuple(0 for _ in a.shape))
    return pl.pallas_call(
        _bro_body,
        grid=(e // be,),
        in_specs=[
            pl.BlockSpec((be, 64), lambda i: (i, 0)),
            pl.BlockSpec((be, 16), lambda i: (i, 0)),
            full(w1e), full(b1), full(w2), full(b2), full(w3), full(b3),
            full(w4), full(b4), full(w5), full(b5),
        ],
        out_specs=pl.BlockSpec((be, 128), lambda i: (i, 0)),
        out_shape=jax.ShapeDtypeStruct((e, 128), jnp.float32),
    )(g, ef, w1e, b1, w2, b2, w3, b3, w4, b4, w5, b5)


def _msg_body(g_ref, ef_ref, w1e_ref, b1_ref, w2_ref, b2_ref, w3_ref, b3_ref,
              w4_ref, b4_ref, o1_ref, o2_ref):
    h = g_ref[:] + jnp.dot(ef_ref[:], w1e_ref[:], preferred_element_type=jnp.float32) + b1_ref[:]
    h = _leaky(h)
    h = _leaky(jnp.dot(h, w2_ref[:], preferred_element_type=jnp.float32) + b2_ref[:])
    h = _leaky(jnp.dot(h, w3_ref[:], preferred_element_type=jnp.float32) + b3_ref[:])
    x = jnp.dot(h, w4_ref[:], preferred_element_type=jnp.float32) + b4_ref[:]
    kk = jax.nn.sigmoid(x[:, :1])
    o1_ref[:] = x[:, 1:33] * kk
    o2_ref[:] = x[:, 33:65] * kk


def _msg_edge_mlp(g, ef, w1e, b1, w2, b2, w3, b3, w4, b4):
    e = g.shape[0]
    be = 3200
    full = lambda a: pl.BlockSpec(a.shape, lambda i: tuple(0 for _ in a.shape))
    return pl.pallas_call(
        _msg_body,
        grid=(e // be,),
        in_specs=[
            pl.BlockSpec((be, 64), lambda i: (i, 0)),
            pl.BlockSpec((be, 16), lambda i: (i, 0)),
            full(w1e), full(b1), full(w2), full(b2), full(w3), full(b3),
            full(w4), full(b4),
        ],
        out_specs=[
            pl.BlockSpec((be, 32), lambda i: (i, 0)),
            pl.BlockSpec((be, 32), lambda i: (i, 0)),
        ],
        out_shape=[
            jax.ShapeDtypeStruct((e, 32), jnp.float32),
            jax.ShapeDtypeStruct((e, 32), jnp.float32),
        ],
    )(g, ef, w1e, b1, w2, b2, w3, b3, w4, b4)


def _red_body(x_ref, w1_ref, b1_ref, w2_ref, b2_ref, w3_ref, b3_ref,
              w4_ref, b4_ref, out_ref):
    h = _leaky(jnp.dot(x_ref[:], w1_ref[:], preferred_element_type=jnp.float32) + b1_ref[:])
    h = _leaky(jnp.dot(h, w2_ref[:], preferred_element_type=jnp.float32) + b2_ref[:])
    h = _leaky(jnp.dot(h, w3_ref[:], preferred_element_type=jnp.float32) + b3_ref[:])
    out_ref[:] = jnp.dot(h, w4_ref[:], preferred_element_type=jnp.float32) + b4_ref[:]


def _red_mlp(x, w1, b1, w2, b2, w3, b3, w4, b4):
    b, k = x.shape
    bb = 1000
    full = lambda a: pl.BlockSpec(a.shape, lambda i: tuple(0 for _ in a.shape))
    return pl.pallas_call(
        _red_body,
        grid=(b // bb,),
        in_specs=[
            pl.BlockSpec((bb, k), lambda i: (i, 0)),
            full(w1), full(b1), full(w2), full(b2), full(w3), full(b3),
            full(w4), full(b4),
        ],
        out_specs=pl.BlockSpec((bb, 128), lambda i: (i, 0)),
        out_shape=jax.ShapeDtypeStruct((b, 128), jnp.float32),
    )(x, w1, b1, w2, b2, w3, b3, w4, b4)


def kernel(nf, edge_index_out, ef_out, edge_index_in, ef_in, output_nodes,
           msg_params, red_params, bro_params):
    n = nf.shape[0]
    f2d = lambda v: v.reshape(1, -1)

    (bw1, bb1), (bw2, bb2), (bw3, bb3), (bw4, bb4), (bw5, bb5) = bro_params
    (mw1, mb1), (mw2, mb2), (mw3, mb3), (mw4, mb4) = msg_params
    (rw1, rb1), (rw2, rb2), (rw3, rb3), (rw4, rb4) = red_params

    wcat = jnp.concatenate(
        [bw1[:128], bw1[128:256], mw1[:128], mw1[128:256]], axis=1)
    proj = _node_proj(nf, wcat)

    src_o, dst_o = edge_index_out[0], edge_index_out[1]
    src_i, dst_i = edge_index_in[0], edge_index_in[1]

    g_a = proj[src_o, 0:64] + proj[dst_o, 64:128]
    g_b = proj[src_i, 128:192] + proj[dst_i, 192:256]

    efi = _bro_edge_mlp(g_a, ef_out, bw1[256:272], f2d(bb1), bw2, f2d(bb2),
                        bw3, f2d(bb3), bw4, f2d(bb4), bw5, f2d(bb5))
    efo1, efo2 = _msg_edge_mlp(g_b, ef_in, mw1[256:272], f2d(mb1), mw2, f2d(mb2),
                               mw3, f2d(mb3), mw4, f2d(mb4))

    new_nf = jax.ops.segment_sum(efi, dst_o, num_segments=n)
    nfo1 = jax.ops.segment_sum(efo1, dst_i, num_segments=n)
    nfo2 = jax.ops.segment_max(efo2, dst_i, num_segments=n)
    nfo2 = jnp.where(jnp.isneginf(nfo2), 0.0, nfo2)

    x = jnp.concatenate(
        [nf[output_nodes], nfo1[output_nodes], nfo2[output_nodes]], axis=1)
    upd = _red_mlp(x, rw1, f2d(rb1), rw2, f2d(rb2), rw3, f2d(rb3), rw4, f2d(rb4))
    return new_nf.at[output_nodes].set(upd)


# speedup vs baseline: 171.5531x; 171.5531x over previous
import functools

import jax
import jax.numpy as jnp
from jax.experimental import pallas as pl
from jax.experimental.pallas import tpu as pltpu


def _leaky(x):
    return jnp.where(x >= 0, x, 0.2 * x)


def _proj_body(nf_ref, w_ref, o0_ref, o1_ref, o2_ref, o3_ref):
    p = jnp.dot(nf_ref[:], w_ref[:], preferred_element_type=jnp.float32)
    o0_ref[:] = p[:, 0:64]
    o1_ref[:] = p[:, 64:128]
    o2_ref[:] = p[:, 128:192]
    o3_ref[:] = p[:, 192:256]


def _node_proj(nf, wcat):
    n, k = nf.shape
    bn = 2000
    out64 = lambda: pl.BlockSpec((bn, 64), lambda i: (i, 0))
    return pl.pallas_call(
        _proj_body,
        grid=(n // bn,),
        in_specs=[
            pl.BlockSpec((bn, k), lambda i: (i, 0)),
            pl.BlockSpec((k, 256), lambda i: (0, 0)),
        ],
        out_specs=[out64(), out64(), out64(), out64()],
        out_shape=[jax.ShapeDtypeStruct((n, 64), jnp.float32)] * 4,
    )(nf, wcat)


def _bro_body(g_ref, ef_ref, w1e_ref, b1_ref, w2_ref, b2_ref, w3_ref, b3_ref,
              w4_ref, b4_ref, w5_ref, b5_ref, out_ref):
    h = g_ref[:] + jnp.dot(ef_ref[:], w1e_ref[:], preferred_element_type=jnp.float32) + b1_ref[:]
    h = _leaky(h)
    h = _leaky(jnp.dot(h, w2_ref[:], preferred_element_type=jnp.float32) + b2_ref[:])
    h = _leaky(jnp.dot(h, w3_ref[:], preferred_element_type=jnp.float32) + b3_ref[:])
    h = _leaky(jnp.dot(h, w4_ref[:], preferred_element_type=jnp.float32) + b4_ref[:])
    out_ref[:] = jnp.dot(h, w5_ref[:], preferred_element_type=jnp.float32) + b5_ref[:]


def _bro_edge_mlp(g, ef, w1e, b1, w2, b2, w3, b3, w4, b4, w5, b5):
    e = g.shape[0]
    be = 3200
    full = lambda a: pl.BlockSpec(a.shape, lambda i: tuple(0 for _ in a.shape))
    return pl.pallas_call(
        _bro_body,
        grid=(e // be,),
        in_specs=[
            pl.BlockSpec((be, 64), lambda i: (i, 0)),
            pl.BlockSpec((be, 16), lambda i: (i, 0)),
            full(w1e), full(b1), full(w2), full(b2), full(w3), full(b3),
            full(w4), full(b4), full(w5), full(b5),
        ],
        out_specs=pl.BlockSpec((be, 128), lambda i: (i, 0)),
        out_shape=jax.ShapeDtypeStruct((e, 128), jnp.float32),
    )(g, ef, w1e, b1, w2, b2, w3, b3, w4, b4, w5, b5)


def _msg_body(g_ref, ef_ref, w1e_ref, b1_ref, w2_ref, b2_ref, w3_ref, b3_ref,
              w4_ref, b4_ref, o1_ref, o2_ref):
    h = g_ref[:] + jnp.dot(ef_ref[:], w1e_ref[:], preferred_element_type=jnp.float32) + b1_ref[:]
    h = _leaky(h)
    h = _leaky(jnp.dot(h, w2_ref[:], preferred_element_type=jnp.float32) + b2_ref[:])
    h = _leaky(jnp.dot(h, w3_ref[:], preferred_element_type=jnp.float32) + b3_ref[:])
    x = jnp.dot(h, w4_ref[:], preferred_element_type=jnp.float32) + b4_ref[:]
    kk = jax.nn.sigmoid(x[:, :1])
    o1_ref[:] = x[:, 1:33] * kk
    o2_ref[:] = x[:, 33:65] * kk


def _msg_edge_mlp(g, ef, w1e, b1, w2, b2, w3, b3, w4, b4):
    e = g.shape[0]
    be = 3200
    full = lambda a: pl.BlockSpec(a.shape, lambda i: tuple(0 for _ in a.shape))
    return pl.pallas_call(
        _msg_body,
        grid=(e // be,),
        in_specs=[
            pl.BlockSpec((be, 64), lambda i: (i, 0)),
            pl.BlockSpec((be, 16), lambda i: (i, 0)),
            full(w1e), full(b1), full(w2), full(b2), full(w3), full(b3),
            full(w4), full(b4),
        ],
        out_specs=[
            pl.BlockSpec((be, 32), lambda i: (i, 0)),
            pl.BlockSpec((be, 32), lambda i: (i, 0)),
        ],
        out_shape=[
            jax.ShapeDtypeStruct((e, 32), jnp.float32),
            jax.ShapeDtypeStruct((e, 32), jnp.float32),
        ],
    )(g, ef, w1e, b1, w2, b2, w3, b3, w4, b4)


def _red_body(x_ref, w1_ref, b1_ref, w2_ref, b2_ref, w3_ref, b3_ref,
              w4_ref, b4_ref, out_ref):
    h = _leaky(jnp.dot(x_ref[:], w1_ref[:], preferred_element_type=jnp.float32) + b1_ref[:])
    h = _leaky(jnp.dot(h, w2_ref[:], preferred_element_type=jnp.float32) + b2_ref[:])
    h = _leaky(jnp.dot(h, w3_ref[:], preferred_element_type=jnp.float32) + b3_ref[:])
    out_ref[:] = jnp.dot(h, w4_ref[:], preferred_element_type=jnp.float32) + b4_ref[:]


def _red_mlp(x, w1, b1, w2, b2, w3, b3, w4, b4):
    b, k = x.shape
    bb = 1000
    full = lambda a: pl.BlockSpec(a.shape, lambda i: tuple(0 for _ in a.shape))
    return pl.pallas_call(
        _red_body,
        grid=(b // bb,),
        in_specs=[
            pl.BlockSpec((bb, k), lambda i: (i, 0)),
            full(w1), full(b1), full(w2), full(b2), full(w3), full(b3),
            full(w4), full(b4),
        ],
        out_specs=pl.BlockSpec((bb, 128), lambda i: (i, 0)),
        out_shape=jax.ShapeDtypeStruct((b, 128), jnp.float32),
    )(x, w1, b1, w2, b2, w3, b3, w4, b4)


def kernel(nf, edge_index_out, ef_out, edge_index_in, ef_in, output_nodes,
           msg_params, red_params, bro_params):
    n = nf.shape[0]
    f2d = lambda v: v.reshape(1, -1)

    (bw1, bb1), (bw2, bb2), (bw3, bb3), (bw4, bb4), (bw5, bb5) = bro_params
    (mw1, mb1), (mw2, mb2), (mw3, mb3), (mw4, mb4) = msg_params
    (rw1, rb1), (rw2, rb2), (rw3, rb3), (rw4, rb4) = red_params

    wcat = jnp.concatenate(
        [bw1[:128], bw1[128:256], mw1[:128], mw1[128:256]], axis=1)
    pa_s, pa_d, pb_s, pb_d = _node_proj(nf, wcat)

    src_o, dst_o = edge_index_out[0], edge_index_out[1]
    src_i, dst_i = edge_index_in[0], edge_index_in[1]

    g_a = pa_s[src_o] + pa_d[dst_o]
    g_b = pb_s[src_i] + pb_d[dst_i]

    efi = _bro_edge_mlp(g_a, ef_out, bw1[256:272], f2d(bb1), bw2, f2d(bb2),
                        bw3, f2d(bb3), bw4, f2d(bb4), bw5, f2d(bb5))
    efo1, efo2 = _msg_edge_mlp(g_b, ef_in, mw1[256:272], f2d(mb1), mw2, f2d(mb2),
                               mw3, f2d(mb3), mw4, f2d(mb4))

    new_nf = jax.ops.segment_sum(efi, dst_o, num_segments=n)
    nfo1 = jax.ops.segment_sum(efo1, dst_i, num_segments=n)
    nfo2 = jax.ops.segment_max(efo2, dst_i, num_segments=n)
    nfo2 = jnp.where(jnp.isneginf(nfo2), 0.0, nfo2)

    x = jnp.concatenate(
        [nf[output_nodes], nfo1[output_nodes], nfo2[output_nodes]], axis=1)
    upd = _red_mlp(x, rw1, f2d(rb1), rw2, f2d(rb2), rw3, f2d(rb3), rw4, f2d(rb4))
    return new_nf.at[output_nodes].set(upd)


# speedup vs baseline: 267.8801x; 1.5615x over previous
import functools

import jax
import jax.numpy as jnp
from jax import lax
from jax.experimental import pallas as pl
from jax.experimental.pallas import tpu as pltpu
from jax.experimental.pallas import tpu_sc as plsc

_NC = 2
_NS = 16
_NW = _NC * _NS


def _leaky(x):
    return jnp.where(x >= 0, x, 0.2 * x)


_GC = 80


def _sc_gather_paths(t_a, t_b, src_o, dst_o, src_i, dst_i):
    e = src_o.shape[0]
    per_w = e // _NW
    n_chunks = per_w // _GC
    mesh = plsc.VectorSubcoreMesh(core_axis_name="c", subcore_axis_name="s")

    def body(ta_h, tb_h, so_h, do_h, si_h, di_h, g_h,
             idx_s, idx_d, buf_s, buf_d, buf_o, sem1, sem2):
        cid = lax.axis_index("c")
        sid = lax.axis_index("s")
        wid = sid * _NC + cid
        base = wid * per_w

        def do_path(t_h, s_h, d_h, col, ci):
            off = base + ci * _GC
            pltpu.sync_copy(s_h.at[pl.ds(off, _GC)], idx_s)
            pltpu.sync_copy(d_h.at[pl.ds(off, _GC)], idx_d)
            cp1 = pltpu.async_copy(t_h.at[idx_s], buf_s, sem1)
            cp2 = pltpu.async_copy(t_h.at[idx_d], buf_d, sem2)
            cp1.wait()
            cp2.wait()

            def row(r, carry):
                for j in range(64 // 16):
                    buf_o[r, pl.ds(col + j * 16, 16)] = (
                        buf_s[r, pl.ds(j * 16, 16)]
                        + buf_d[r, pl.ds(64 + j * 16, 16)])
                return carry
            lax.fori_loop(0, _GC, row, 0)

        def step(ci, carry):
            do_path(ta_h, so_h, do_h, 0, ci)
            do_path(tb_h, si_h, di_h, 64, ci)
            pltpu.sync_copy(buf_o, g_h.at[pl.ds(base + ci * _GC, _GC)])
            return carry
        lax.fori_loop(0, n_chunks, step, 0)

    f = pl.kernel(
        body,
        out_type=jax.ShapeDtypeStruct((e, 128), jnp.float32),
        mesh=mesh,
        scratch_types=[
            pltpu.VMEM((_GC,), jnp.int32),
            pltpu.VMEM((_GC,), jnp.int32),
            pltpu.VMEM((_GC, 128), jnp.float32),
            pltpu.VMEM((_GC, 128), jnp.float32),
            pltpu.VMEM((_GC, 128), jnp.float32),
            pltpu.SemaphoreType.DMA,
            pltpu.SemaphoreType.DMA,
        ],
    )
    return f(t_a, t_b, src_o, dst_o, src_i, dst_i)


def _proj_body(nf_ref, w_ref, o0_ref, o1_ref):
    p = jnp.dot(nf_ref[:], w_ref[:], preferred_element_type=jnp.float32)
    o0_ref[:] = p[:, 0:128]
    o1_ref[:] = p[:, 128:256]


def _node_proj(nf, wcat):
    n, k = nf.shape
    bn = 2000
    out128 = lambda: pl.BlockSpec((bn, 128), lambda i: (i, 0))
    return pl.pallas_call(
        _proj_body,
        grid=(n // bn,),
        in_specs=[
            pl.BlockSpec((bn, k), lambda i: (i, 0)),
            pl.BlockSpec((k, 256), lambda i: (0, 0)),
        ],
        out_specs=[out128(), out128()],
        out_shape=[jax.ShapeDtypeStruct((n, 128), jnp.float32)] * 2,
    )(nf, wcat)


def _bro_body(g_ref, ef_ref, w1e_ref, b1_ref, w2_ref, b2_ref, w3_ref, b3_ref,
              w4_ref, b4_ref, w5_ref, b5_ref, out_ref):
    h = g_ref[:, 0:64] + jnp.dot(ef_ref[:], w1e_ref[:], preferred_element_type=jnp.float32) + b1_ref[:]
    h = _leaky(h)
    h = _leaky(jnp.dot(h, w2_ref[:], preferred_element_type=jnp.float32) + b2_ref[:])
    h = _leaky(jnp.dot(h, w3_ref[:], preferred_element_type=jnp.float32) + b3_ref[:])
    h = _leaky(jnp.dot(h, w4_ref[:], preferred_element_type=jnp.float32) + b4_ref[:])
    out_ref[:] = jnp.dot(h, w5_ref[:], preferred_element_type=jnp.float32) + b5_ref[:]


def _bro_edge_mlp(g, ef, w1e, b1, w2, b2, w3, b3, w4, b4, w5, b5):
    e = g.shape[0]
    be = 3200
    full = lambda a: pl.BlockSpec(a.shape, lambda i: tuple(0 for _ in a.shape))
    return pl.pallas_call(
        _bro_body,
        grid=(e // be,),
        in_specs=[
            pl.BlockSpec((be, 128), lambda i: (i, 0)),
            pl.BlockSpec((be, 16), lambda i: (i, 0)),
            full(w1e), full(b1), full(w2), full(b2), full(w3), full(b3),
            full(w4), full(b4), full(w5), full(b5),
        ],
        out_specs=pl.BlockSpec((be, 128), lambda i: (i, 0)),
        out_shape=jax.ShapeDtypeStruct((e, 128), jnp.float32),
    )(g, ef, w1e, b1, w2, b2, w3, b3, w4, b4, w5, b5)


def _msg_body(g_ref, ef_ref, w1e_ref, b1_ref, w2_ref, b2_ref, w3_ref, b3_ref,
              w4_ref, b4_ref, o1_ref, o2_ref):
    h = g_ref[:, 64:128] + jnp.dot(ef_ref[:], w1e_ref[:], preferred_element_type=jnp.float32) + b1_ref[:]
    h = _leaky(h)
    h = _leaky(jnp.dot(h, w2_ref[:], preferred_element_type=jnp.float32) + b2_ref[:])
    h = _leaky(jnp.dot(h, w3_ref[:], preferred_element_type=jnp.float32) + b3_ref[:])
    x = jnp.dot(h, w4_ref[:], preferred_element_type=jnp.float32) + b4_ref[:]
    kk = jax.nn.sigmoid(x[:, :1])
    o1_ref[:] = x[:, 1:33] * kk
    o2_ref[:] = x[:, 33:65] * kk


def _msg_edge_mlp(g, ef, w1e, b1, w2, b2, w3, b3, w4, b4):
    e = g.shape[0]
    be = 3200
    full = lambda a: pl.BlockSpec(a.shape, lambda i: tuple(0 for _ in a.shape))
    return pl.pallas_call(
        _msg_body,
        grid=(e // be,),
        in_specs=[
            pl.BlockSpec((be, 128), lambda i: (i, 0)),
            pl.BlockSpec((be, 16), lambda i: (i, 0)),
            full(w1e), full(b1), full(w2), full(b2), full(w3), full(b3),
            full(w4), full(b4),
        ],
        out_specs=[
            pl.BlockSpec((be, 32), lambda i: (i, 0)),
            pl.BlockSpec((be, 32), lambda i: (i, 0)),
        ],
        out_shape=[
            jax.ShapeDtypeStruct((e, 32), jnp.float32),
            jax.ShapeDtypeStruct((e, 32), jnp.float32),
        ],
    )(g, ef, w1e, b1, w2, b2, w3, b3, w4, b4)


def _red_body(x_ref, w1_ref, b1_ref, w2_ref, b2_ref, w3_ref, b3_ref,
              w4_ref, b4_ref, out_ref):
    h = _leaky(jnp.dot(x_ref[:], w1_ref[:], preferred_element_type=jnp.float32) + b1_ref[:])
    h = _leaky(jnp.dot(h, w2_ref[:], preferred_element_type=jnp.float32) + b2_ref[:])
    h = _leaky(jnp.dot(h, w3_ref[:], preferred_element_type=jnp.float32) + b3_ref[:])
    out_ref[:] = jnp.dot(h, w4_ref[:], preferred_element_type=jnp.float32) + b4_ref[:]


def _red_mlp(x, w1, b1, w2, b2, w3, b3, w4, b4):
    b, k = x.shape
    bb = 1000
    full = lambda a: pl.BlockSpec(a.shape, lambda i: tuple(0 for _ in a.shape))
    return pl.pallas_call(
        _red_body,
        grid=(b // bb,),
        in_specs=[
            pl.BlockSpec((bb, k), lambda i: (i, 0)),
            full(w1), full(b1), full(w2), full(b2), full(w3), full(b3),
            full(w4), full(b4),
        ],
        out_specs=pl.BlockSpec((bb, 128), lambda i: (i, 0)),
        out_shape=jax.ShapeDtypeStruct((b, 128), jnp.float32),
    )(x, w1, b1, w2, b2, w3, b3, w4, b4)


def kernel(nf, edge_index_out, ef_out, edge_index_in, ef_in, output_nodes,
           msg_params, red_params, bro_params):
    n = nf.shape[0]
    f2d = lambda v: v.reshape(1, -1)

    (bw1, bb1), (bw2, bb2), (bw3, bb3), (bw4, bb4), (bw5, bb5) = bro_params
    (mw1, mb1), (mw2, mb2), (mw3, mb3), (mw4, mb4) = msg_params
    (rw1, rb1), (rw2, rb2), (rw3, rb3), (rw4, rb4) = red_params

    wcat = jnp.concatenate(
        [bw1[:128], bw1[128:256], mw1[:128], mw1[128:256]], axis=1)
    t_a, t_b = _node_proj(nf, wcat)

    src_o, dst_o = edge_index_out[0], edge_index_out[1]
    src_i, dst_i = edge_index_in[0], edge_index_in[1]

    g = _sc_gather_paths(t_a, t_b, src_o, dst_o, src_i, dst_i)

    efi = _bro_edge_mlp(g, ef_out, bw1[256:272], f2d(bb1), bw2, f2d(bb2),
                        bw3, f2d(bb3), bw4, f2d(bb4), bw5, f2d(bb5))
    efo1, efo2 = _msg_edge_mlp(g, ef_in, mw1[256:272], f2d(mb1), mw2, f2d(mb2),
                               mw3, f2d(mb3), mw4, f2d(mb4))

    new_nf = jax.ops.segment_sum(efi, dst_o, num_segments=n)
    nfo1 = jax.ops.segment_sum(efo1, dst_i, num_segments=n)
    nfo2 = jax.ops.segment_max(efo2, dst_i, num_segments=n)
    nfo2 = jnp.where(jnp.isneginf(nfo2), 0.0, nfo2)

    x = jnp.concatenate(
        [nf[output_nodes], nfo1[output_nodes], nfo2[output_nodes]], axis=1)
    upd = _red_mlp(x, rw1, f2d(rb1), rw2, f2d(rb2), rw3, f2d(rb3), rw4, f2d(rb4))
    return new_nf.at[output_nodes].set(upd)


# speedup vs baseline: 313.1918x; 1.1691x over previous
import functools

import jax
import jax.numpy as jnp
from jax import lax
from jax.experimental import pallas as pl
from jax.experimental.pallas import tpu as pltpu
from jax.experimental.pallas import tpu_sc as plsc

_NC = 2
_NS = 16
_NW = _NC * _NS


def _leaky(x):
    return jnp.where(x >= 0, x, 0.2 * x)


_GC = 80


def _sc_gather_paths(t_a, t_b, src_o, dst_o, src_i, dst_i):
    e = src_o.shape[0]
    per_w = e // _NW
    n_chunks = per_w // _GC
    mesh = plsc.VectorSubcoreMesh(core_axis_name="c", subcore_axis_name="s")

    def body(ta_h, tb_h, so_h, do_h, si_h, di_h, g_h,
             idx_s, idx_d, buf_s, buf_d, buf_o, sem1, sem2):
        cid = lax.axis_index("c")
        sid = lax.axis_index("s")
        wid = sid * _NC + cid
        base = wid * per_w

        def do_path(t_h, s_h, d_h, col, ci):
            off = base + ci * _GC
            pltpu.sync_copy(s_h.at[pl.ds(off, _GC)], idx_s)
            pltpu.sync_copy(d_h.at[pl.ds(off, _GC)], idx_d)
            cp1 = pltpu.async_copy(t_h.at[idx_s], buf_s, sem1)
            cp2 = pltpu.async_copy(t_h.at[idx_d], buf_d, sem2)
            cp1.wait()
            cp2.wait()

            def row(r, carry):
                for j in range(64 // 16):
                    buf_o[r, pl.ds(col + j * 16, 16)] = (
                        buf_s[r, pl.ds(j * 16, 16)]
                        + buf_d[r, pl.ds(64 + j * 16, 16)])
                return carry
            lax.fori_loop(0, _GC, row, 0)

        def step(ci, carry):
            do_path(ta_h, so_h, do_h, 0, ci)
            do_path(tb_h, si_h, di_h, 64, ci)
            pltpu.sync_copy(buf_o, g_h.at[pl.ds(base + ci * _GC, _GC)])
            return carry
        lax.fori_loop(0, n_chunks, step, 0)

    f = pl.kernel(
        body,
        out_type=jax.ShapeDtypeStruct((e, 128), jnp.float32),
        mesh=mesh,
        scratch_types=[
            pltpu.VMEM((_GC,), jnp.int32),
            pltpu.VMEM((_GC,), jnp.int32),
            pltpu.VMEM((_GC, 128), jnp.float32),
            pltpu.VMEM((_GC, 128), jnp.float32),
            pltpu.VMEM((_GC, 128), jnp.float32),
            pltpu.SemaphoreType.DMA,
            pltpu.SemaphoreType.DMA,
        ],
    )
    return f(t_a, t_b, src_o, dst_o, src_i, dst_i)


def _proj_body(nf_ref, w_ref, o0_ref, o1_ref):
    p = jnp.dot(nf_ref[:], w_ref[:], preferred_element_type=jnp.float32)
    o0_ref[:] = p[:, 0:128]
    o1_ref[:] = p[:, 128:256]


def _node_proj(nf, wcat):
    n, k = nf.shape
    bn = 2000
    out128 = lambda: pl.BlockSpec((bn, 128), lambda i: (i, 0))
    return pl.pallas_call(
        _proj_body,
        grid=(n // bn,),
        in_specs=[
            pl.BlockSpec((bn, k), lambda i: (i, 0)),
            pl.BlockSpec((k, 256), lambda i: (0, 0)),
        ],
        out_specs=[out128(), out128()],
        out_shape=[jax.ShapeDtypeStruct((n, 128), jnp.float32)] * 2,
    )(nf, wcat)


def _sc_segment_sum_efi(efi, dst_o, n):
    e = dst_o.shape[0]
    per_s = e // _NS
    half = n // 2
    rows_w = 1000
    mesh = plsc.VectorSubcoreMesh(core_axis_name="c", subcore_axis_name="s")

    def body(efi_h, do_h, out_h, idx_a, rows_a, zba, acc_a, sem):
        cid = lax.axis_index("c")
        sid = lax.axis_index("s")
        base = cid * half

        def zrow(r, carry):
            for j in range(128 // 16):
                zba[r, pl.ds(j * 16, 16)] = jnp.zeros((16,), jnp.float32)
            return carry
        lax.fori_loop(0, 200, zrow, 0)

        @pl.when(sid < half // rows_w)
        def _():
            def zcp(k, carry):
                off = sid * rows_w + k * 200
                pltpu.sync_copy(zba, acc_a.at[pl.ds(off, 200)])
                return carry
            lax.fori_loop(0, rows_w // 200, zcp, 0)
        plsc.subcore_barrier()

        def step(ci, carry):
            off = sid * per_s + ci * _GC
            pltpu.sync_copy(do_h.at[pl.ds(off, _GC)], idx_a)
            pltpu.sync_copy(efi_h.at[pl.ds(off, _GC)], rows_a)
            for k in range(_GC // 16):
                sl = pl.ds(k * 16, 16)
                v = idx_a[sl] - base
                inb = (v >= 0) & (v < half)
                idx_a[sl] = jnp.where(inb, v, half)
            pltpu.sync_copy(rows_a, acc_a.at[idx_a], add=True)
            return carry
        lax.fori_loop(0, per_s // _GC, step, 0)
        plsc.subcore_barrier()

        @pl.when(sid < half // rows_w)
        def _():
            off = sid * rows_w
            pltpu.sync_copy(acc_a.at[pl.ds(off, rows_w)],
                            out_h.at[pl.ds(base + off, rows_w)])

    f = pl.kernel(
        body,
        out_type=jax.ShapeDtypeStruct((n, 128), jnp.float32),
        mesh=mesh,
        scratch_types=[
            pltpu.VMEM((_GC,), jnp.int32),
            pltpu.VMEM((_GC, 128), jnp.float32),
            pltpu.VMEM((200, 128), jnp.float32),
            pltpu.VMEM_SHARED((half + 8, 128), jnp.float32),
            pltpu.SemaphoreType.DMA,
        ],
    )
    return f(efi, dst_o)


def _bro_body(g_ref, ef_ref, w1e_ref, b1_ref, w2_ref, b2_ref, w3_ref, b3_ref,
              w4_ref, b4_ref, w5_ref, b5_ref, out_ref):
    h = g_ref[:, 0:64] + jnp.dot(ef_ref[:], w1e_ref[:], preferred_element_type=jnp.float32) + b1_ref[:]
    h = _leaky(h)
    h = _leaky(jnp.dot(h, w2_ref[:], preferred_element_type=jnp.float32) + b2_ref[:])
    h = _leaky(jnp.dot(h, w3_ref[:], preferred_element_type=jnp.float32) + b3_ref[:])
    h = _leaky(jnp.dot(h, w4_ref[:], preferred_element_type=jnp.float32) + b4_ref[:])
    out_ref[:] = jnp.dot(h, w5_ref[:], preferred_element_type=jnp.float32) + b5_ref[:]


def _bro_edge_mlp(g, ef, w1e, b1, w2, b2, w3, b3, w4, b4, w5, b5):
    e = g.shape[0]
    be = 3200
    full = lambda a: pl.BlockSpec(a.shape, lambda i: tuple(0 for _ in a.shape))
    return pl.pallas_call(
        _bro_body,
        grid=(e // be,),
        in_specs=[
            pl.BlockSpec((be, 128), lambda i: (i, 0)),
            pl.BlockSpec((be, 16), lambda i: (i, 0)),
            full(w1e), full(b1), full(w2), full(b2), full(w3), full(b3),
            full(w4), full(b4), full(w5), full(b5),
        ],
        out_specs=pl.BlockSpec((be, 128), lambda i: (i, 0)),
        out_shape=jax.ShapeDtypeStruct((e, 128), jnp.float32),
    )(g, ef, w1e, b1, w2, b2, w3, b3, w4, b4, w5, b5)


def _msg_body(g_ref, ef_ref, w1e_ref, b1_ref, w2_ref, b2_ref, w3_ref, b3_ref,
              w4_ref, b4_ref, o1_ref, o2_ref):
    h = g_ref[:, 64:128] + jnp.dot(ef_ref[:], w1e_ref[:], preferred_element_type=jnp.float32) + b1_ref[:]
    h = _leaky(h)
    h = _leaky(jnp.dot(h, w2_ref[:], preferred_element_type=jnp.float32) + b2_ref[:])
    h = _leaky(jnp.dot(h, w3_ref[:], preferred_element_type=jnp.float32) + b3_ref[:])
    x = jnp.dot(h, w4_ref[:], preferred_element_type=jnp.float32) + b4_ref[:]
    kk = jax.nn.sigmoid(x[:, :1])
    o1_ref[:] = x[:, 1:33] * kk
    o2_ref[:] = x[:, 33:65] * kk


def _msg_edge_mlp(g, ef, w1e, b1, w2, b2, w3, b3, w4, b4):
    e = g.shape[0]
    be = 3200
    full = lambda a: pl.BlockSpec(a.shape, lambda i: tuple(0 for _ in a.shape))
    return pl.pallas_call(
        _msg_body,
        grid=(e // be,),
        in_specs=[
            pl.BlockSpec((be, 128), lambda i: (i, 0)),
            pl.BlockSpec((be, 16), lambda i: (i, 0)),
            full(w1e), full(b1), full(w2), full(b2), full(w3), full(b3),
            full(w4), full(b4),
        ],
        out_specs=[
            pl.BlockSpec((be, 32), lambda i: (i, 0)),
            pl.BlockSpec((be, 32), lambda i: (i, 0)),
        ],
        out_shape=[
            jax.ShapeDtypeStruct((e, 32), jnp.float32),
            jax.ShapeDtypeStruct((e, 32), jnp.float32),
        ],
    )(g, ef, w1e, b1, w2, b2, w3, b3, w4, b4)


def _red_body(x_ref, w1_ref, b1_ref, w2_ref, b2_ref, w3_ref, b3_ref,
              w4_ref, b4_ref, out_ref):
    h = _leaky(jnp.dot(x_ref[:], w1_ref[:], preferred_element_type=jnp.float32) + b1_ref[:])
    h = _leaky(jnp.dot(h, w2_ref[:], preferred_element_type=jnp.float32) + b2_ref[:])
    h = _leaky(jnp.dot(h, w3_ref[:], preferred_element_type=jnp.float32) + b3_ref[:])
    out_ref[:] = jnp.dot(h, w4_ref[:], preferred_element_type=jnp.float32) + b4_ref[:]


def _red_mlp(x, w1, b1, w2, b2, w3, b3, w4, b4):
    b, k = x.shape
    bb = 1000
    full = lambda a: pl.BlockSpec(a.shape, lambda i: tuple(0 for _ in a.shape))
    return pl.pallas_call(
        _red_body,
        grid=(b // bb,),
        in_specs=[
            pl.BlockSpec((bb, k), lambda i: (i, 0)),
            full(w1), full(b1), full(w2), full(b2), full(w3), full(b3),
            full(w4), full(b4),
        ],
        out_specs=pl.BlockSpec((bb, 128), lambda i: (i, 0)),
        out_shape=jax.ShapeDtypeStruct((b, 128), jnp.float32),
    )(x, w1, b1, w2, b2, w3, b3, w4, b4)


def kernel(nf, edge_index_out, ef_out, edge_index_in, ef_in, output_nodes,
           msg_params, red_params, bro_params):
    n = nf.shape[0]
    f2d = lambda v: v.reshape(1, -1)

    (bw1, bb1), (bw2, bb2), (bw3, bb3), (bw4, bb4), (bw5, bb5) = bro_params
    (mw1, mb1), (mw2, mb2), (mw3, mb3), (mw4, mb4) = msg_params
    (rw1, rb1), (rw2, rb2), (rw3, rb3), (rw4, rb4) = red_params

    wcat = jnp.concatenate(
        [bw1[:128], bw1[128:256], mw1[:128], mw1[128:256]], axis=1)
    t_a, t_b = _node_proj(nf, wcat)

    src_o, dst_o = edge_index_out[0], edge_index_out[1]
    src_i, dst_i = edge_index_in[0], edge_index_in[1]

    g = _sc_gather_paths(t_a, t_b, src_o, dst_o, src_i, dst_i)

    efi = _bro_edge_mlp(g, ef_out, bw1[256:272], f2d(bb1), bw2, f2d(bb2),
                        bw3, f2d(bb3), bw4, f2d(bb4), bw5, f2d(bb5))
    efo1, efo2 = _msg_edge_mlp(g, ef_in, mw1[256:272], f2d(mb1), mw2, f2d(mb2),
                               mw3, f2d(mb3), mw4, f2d(mb4))

    new_nf = _sc_segment_sum_efi(efi, dst_o, n)
    nfo1 = jax.ops.segment_sum(efo1, dst_i, num_segments=n)
    nfo2 = jax.ops.segment_max(efo2, dst_i, num_segments=n)
    nfo2 = jnp.where(jnp.isneginf(nfo2), 0.0, nfo2)

    x = jnp.concatenate(
        [nf[output_nodes], nfo1[output_nodes], nfo2[output_nodes]], axis=1)
    upd = _red_mlp(x, rw1, f2d(rb1), rw2, f2d(rb2), rw3, f2d(rb3), rw4, f2d(rb4))
    return new_nf.at[output_nodes].set(upd)


# speedup vs baseline: 328.1073x; 1.0476x over previous
import functools

import jax
import jax.numpy as jnp
from jax import lax
from jax.experimental import pallas as pl
from jax.experimental.pallas import tpu as pltpu
from jax.experimental.pallas import tpu_sc as plsc

_NC = 2
_NS = 16
_NW = _NC * _NS


def _leaky(x):
    return jnp.where(x >= 0, x, 0.2 * x)


_GC = 80


def _sc_gather_paths(t_a, t_b, src_o, dst_o, src_i, dst_i):
    e = src_o.shape[0]
    per_w = e // _NW
    n_chunks = per_w // _GC
    mesh = plsc.VectorSubcoreMesh(core_axis_name="c", subcore_axis_name="s")

    def body(ta_h, tb_h, so_h, do_h, si_h, di_h, g_h,
             idx_s, idx_d, buf_s, buf_d, buf_o, sem1, sem2):
        cid = lax.axis_index("c")
        sid = lax.axis_index("s")
        wid = sid * _NC + cid
        base = wid * per_w

        def do_path(t_h, s_h, d_h, col, ci):
            off = base + ci * _GC
            pltpu.sync_copy(s_h.at[pl.ds(off, _GC)], idx_s)
            pltpu.sync_copy(d_h.at[pl.ds(off, _GC)], idx_d)
            cp1 = pltpu.async_copy(t_h.at[idx_s], buf_s, sem1)
            cp2 = pltpu.async_copy(t_h.at[idx_d], buf_d, sem2)
            cp1.wait()
            cp2.wait()

            def row(r, carry):
                for j in range(64 // 16):
                    buf_o[r, pl.ds(col + j * 16, 16)] = (
                        buf_s[r, pl.ds(j * 16, 16)]
                        + buf_d[r, pl.ds(64 + j * 16, 16)])
                return carry
            lax.fori_loop(0, _GC, row, 0)

        def step(ci, carry):
            do_path(ta_h, so_h, do_h, 0, ci)
            do_path(tb_h, si_h, di_h, 64, ci)
            pltpu.sync_copy(buf_o, g_h.at[pl.ds(base + ci * _GC, _GC)])
            return carry
        lax.fori_loop(0, n_chunks, step, 0)

    f = pl.kernel(
        body,
        out_type=jax.ShapeDtypeStruct((e, 128), jnp.float32),
        mesh=mesh,
        scratch_types=[
            pltpu.VMEM((_GC,), jnp.int32),
            pltpu.VMEM((_GC,), jnp.int32),
            pltpu.VMEM((_GC, 128), jnp.float32),
            pltpu.VMEM((_GC, 128), jnp.float32),
            pltpu.VMEM((_GC, 128), jnp.float32),
            pltpu.SemaphoreType.DMA,
            pltpu.SemaphoreType.DMA,
        ],
    )
    return f(t_a, t_b, src_o, dst_o, src_i, dst_i)


def _proj_body(nf_ref, w_ref, o0_ref, o1_ref):
    p = jnp.dot(nf_ref[:], w_ref[:], preferred_element_type=jnp.float32)
    o0_ref[:] = p[:, 0:128]
    o1_ref[:] = p[:, 128:256]


def _node_proj(nf, wcat):
    n, k = nf.shape
    bn = 2000
    out128 = lambda: pl.BlockSpec((bn, 128), lambda i: (i, 0))
    return pl.pallas_call(
        _proj_body,
        grid=(n // bn,),
        in_specs=[
            pl.BlockSpec((bn, k), lambda i: (i, 0)),
            pl.BlockSpec((k, 256), lambda i: (0, 0)),
        ],
        out_specs=[out128(), out128()],
        out_shape=[jax.ShapeDtypeStruct((n, 128), jnp.float32)] * 2,
    )(nf, wcat)


def _sc_segment_sum_efi(efi, dst_o, n):
    e = dst_o.shape[0]
    per_s = e // _NS
    half = n // 2
    rows_w = 1000
    mesh = plsc.VectorSubcoreMesh(core_axis_name="c", subcore_axis_name="s")

    def body(efi_h, do_h, out_h, idx_a, rows_a, zba, acc_a, sem):
        cid = lax.axis_index("c")
        sid = lax.axis_index("s")
        base = cid * half

        def zrow(r, carry):
            for j in range(128 // 16):
                zba[r, pl.ds(j * 16, 16)] = jnp.zeros((16,), jnp.float32)
            return carry
        lax.fori_loop(0, 200, zrow, 0)

        @pl.when(sid < half // rows_w)
        def _():
            def zcp(k, carry):
                off = sid * rows_w + k * 200
                pltpu.sync_copy(zba, acc_a.at[pl.ds(off, 200)])
                return carry
            lax.fori_loop(0, rows_w // 200, zcp, 0)
        plsc.subcore_barrier()

        def step(ci, carry):
            off = sid * per_s + ci * _GC
            pltpu.sync_copy(do_h.at[pl.ds(off, _GC)], idx_a)
            pltpu.sync_copy(efi_h.at[pl.ds(off, _GC)], rows_a)
            for k in range(_GC // 16):
                sl = pl.ds(k * 16, 16)
                v = idx_a[sl] - base
                inb = (v >= 0) & (v < half)
                idx_a[sl] = jnp.where(inb, v, half)
            pltpu.sync_copy(rows_a, acc_a.at[idx_a], add=True)
            return carry
        lax.fori_loop(0, per_s // _GC, step, 0)
        plsc.subcore_barrier()

        @pl.when(sid < half // rows_w)
        def _():
            off = sid * rows_w
            pltpu.sync_copy(acc_a.at[pl.ds(off, rows_w)],
                            out_h.at[pl.ds(base + off, rows_w)])

    f = pl.kernel(
        body,
        out_type=jax.ShapeDtypeStruct((n, 128), jnp.float32),
        mesh=mesh,
        scratch_types=[
            pltpu.VMEM((_GC,), jnp.int32),
            pltpu.VMEM((_GC, 128), jnp.float32),
            pltpu.VMEM((200, 128), jnp.float32),
            pltpu.VMEM_SHARED((half + 8, 128), jnp.float32),
            pltpu.SemaphoreType.DMA,
        ],
    )
    return f(efi, dst_o)


def _bro_body(g_ref, ef_ref, w1e_ref, b1_ref, w2_ref, b2_ref, w3_ref, b3_ref,
              w4_ref, b4_ref, w5_ref, b5_ref, out_ref):
    h = g_ref[:, 0:64] + jnp.dot(ef_ref[:], w1e_ref[:], preferred_element_type=jnp.float32) + b1_ref[:]
    h = _leaky(h)
    h = _leaky(jnp.dot(h, w2_ref[:], preferred_element_type=jnp.float32) + b2_ref[:])
    h = _leaky(jnp.dot(h, w3_ref[:], preferred_element_type=jnp.float32) + b3_ref[:])
    h = _leaky(jnp.dot(h, w4_ref[:], preferred_element_type=jnp.float32) + b4_ref[:])
    out_ref[:] = jnp.dot(h, w5_ref[:], preferred_element_type=jnp.float32) + b5_ref[:]


def _bro_edge_mlp(g, ef, w1e, b1, w2, b2, w3, b3, w4, b4, w5, b5):
    e = g.shape[0]
    be = 3200
    full = lambda a: pl.BlockSpec(a.shape, lambda i: tuple(0 for _ in a.shape))
    return pl.pallas_call(
        _bro_body,
        grid=(e // be,),
        in_specs=[
            pl.BlockSpec((be, 128), lambda i: (i, 0)),
            pl.BlockSpec((be, 16), lambda i: (i, 0)),
            full(w1e), full(b1), full(w2), full(b2), full(w3), full(b3),
            full(w4), full(b4), full(w5), full(b5),
        ],
        out_specs=pl.BlockSpec((be, 128), lambda i: (i, 0)),
        out_shape=jax.ShapeDtypeStruct((e, 128), jnp.float32),
    )(g, ef, w1e, b1, w2, b2, w3, b3, w4, b4, w5, b5)


def _msg_body(g_ref, ef_ref, w1e_ref, b1_ref, w2_ref, b2_ref, w3_ref, b3_ref,
              w4_ref, b4_ref, o1_ref, o2_ref):
    h = g_ref[:, 64:128] + jnp.dot(ef_ref[:], w1e_ref[:], preferred_element_type=jnp.float32) + b1_ref[:]
    h = _leaky(h)
    h = _leaky(jnp.dot(h, w2_ref[:], preferred_element_type=jnp.float32) + b2_ref[:])
    h = _leaky(jnp.dot(h, w3_ref[:], preferred_element_type=jnp.float32) + b3_ref[:])
    x = jnp.dot(h, w4_ref[:], preferred_element_type=jnp.float32) + b4_ref[:]
    kk = jax.nn.sigmoid(x[:, :1])
    gated = x[:, 1:65] * kk
    o1_ref[:] = jnp.concatenate(
        [gated[:, 0:32], jnp.zeros((gated.shape[0], 96), jnp.float32)], axis=1)
    o2_ref[:] = gated[:, 32:64]


def _msg_edge_mlp(g, ef, w1e, b1, w2, b2, w3, b3, w4, b4):
    e = g.shape[0]
    be = 3200
    full = lambda a: pl.BlockSpec(a.shape, lambda i: tuple(0 for _ in a.shape))
    return pl.pallas_call(
        _msg_body,
        grid=(e // be,),
        in_specs=[
            pl.BlockSpec((be, 128), lambda i: (i, 0)),
            pl.BlockSpec((be, 16), lambda i: (i, 0)),
            full(w1e), full(b1), full(w2), full(b2), full(w3), full(b3),
            full(w4), full(b4),
        ],
        out_specs=[
            pl.BlockSpec((be, 128), lambda i: (i, 0)),
            pl.BlockSpec((be, 32), lambda i: (i, 0)),
        ],
        out_shape=[
            jax.ShapeDtypeStruct((e, 128), jnp.float32),
            jax.ShapeDtypeStruct((e, 32), jnp.float32),
        ],
    )(g, ef, w1e, b1, w2, b2, w3, b3, w4, b4)


def _red_body(x_ref, w1_ref, b1_ref, w2_ref, b2_ref, w3_ref, b3_ref,
              w4_ref, b4_ref, out_ref):
    h = _leaky(jnp.dot(x_ref[:], w1_ref[:], preferred_element_type=jnp.float32) + b1_ref[:])
    h = _leaky(jnp.dot(h, w2_ref[:], preferred_element_type=jnp.float32) + b2_ref[:])
    h = _leaky(jnp.dot(h, w3_ref[:], preferred_element_type=jnp.float32) + b3_ref[:])
    out_ref[:] = jnp.dot(h, w4_ref[:], preferred_element_type=jnp.float32) + b4_ref[:]


def _red_mlp(x, w1, b1, w2, b2, w3, b3, w4, b4):
    b, k = x.shape
    bb = 1000
    full = lambda a: pl.BlockSpec(a.shape, lambda i: tuple(0 for _ in a.shape))
    return pl.pallas_call(
        _red_body,
        grid=(b // bb,),
        in_specs=[
            pl.BlockSpec((bb, k), lambda i: (i, 0)),
            full(w1), full(b1), full(w2), full(b2), full(w3), full(b3),
            full(w4), full(b4),
        ],
        out_specs=pl.BlockSpec((bb, 128), lambda i: (i, 0)),
        out_shape=jax.ShapeDtypeStruct((b, 128), jnp.float32),
    )(x, w1, b1, w2, b2, w3, b3, w4, b4)


def kernel(nf, edge_index_out, ef_out, edge_index_in, ef_in, output_nodes,
           msg_params, red_params, bro_params):
    n = nf.shape[0]
    f2d = lambda v: v.reshape(1, -1)

    (bw1, bb1), (bw2, bb2), (bw3, bb3), (bw4, bb4), (bw5, bb5) = bro_params
    (mw1, mb1), (mw2, mb2), (mw3, mb3), (mw4, mb4) = msg_params
    (rw1, rb1), (rw2, rb2), (rw3, rb3), (rw4, rb4) = red_params

    wcat = jnp.concatenate(
        [bw1[:128], bw1[128:256], mw1[:128], mw1[128:256]], axis=1)
    t_a, t_b = _node_proj(nf, wcat)

    src_o, dst_o = edge_index_out[0], edge_index_out[1]
    src_i, dst_i = edge_index_in[0], edge_index_in[1]

    g = _sc_gather_paths(t_a, t_b, src_o, dst_o, src_i, dst_i)

    efi = _bro_edge_mlp(g, ef_out, bw1[256:272], f2d(bb1), bw2, f2d(bb2),
                        bw3, f2d(bb3), bw4, f2d(bb4), bw5, f2d(bb5))
    efo1, efo2 = _msg_edge_mlp(g, ef_in, mw1[256:272], f2d(mb1), mw2, f2d(mb2),
                               mw3, f2d(mb3), mw4, f2d(mb4))

    new_nf = _sc_segment_sum_efi(efi, dst_o, n)
    nfo1_full = _sc_segment_sum_efi(efo1, dst_i, n)
    nfo2 = jax.ops.segment_max(efo2, dst_i, num_segments=n)
    nfo2 = jnp.where(jnp.isneginf(nfo2), 0.0, nfo2)

    x = jnp.concatenate(
        [nf[output_nodes], nfo1_full[output_nodes][:, 0:32],
         nfo2[output_nodes]], axis=1)
    upd = _red_mlp(x, rw1, f2d(rb1), rw2, f2d(rb2), rw3, f2d(rb3), rw4, f2d(rb4))
    return new_nf.at[output_nodes].set(upd)
